# baseline (speedup 1.0000x reference)
import functools

import jax
import jax.numpy as jnp
from jax import lax
from jax.experimental import pallas as pl
from jax.experimental.pallas import tpu as pltpu
from jax.experimental.pallas import tpu_sc as plsc

B = 8192
K = 64
DX = 128
DS = 64
DH = 1024
NSN = 2

BLK = 2048
NBLK = B // BLK
BLKH = 1024
NBLKH = B // BLKH
VB_ROWS = K + 8


def _tc1_body(x_ref, centers_ref, sn_W_ref, sn_b_ref, rwt_ref,
              cid_ref, vb_ref):
    xb = x_ref[...]
    c = centers_ref[...]
    cs = jnp.sum(c * c, axis=1)
    xc_t = lax.dot_general(c, xb, (((1,), (1,)), ((), ())),
                           preferred_element_type=jnp.float32)
    d2_t = cs[:, None] - 2.0 * xc_t
    m = jnp.min(d2_t, axis=0)
    ids = lax.broadcasted_iota(jnp.int32, d2_t.shape, 0)
    cid_ref[...] = jnp.min(jnp.where(d2_t <= m[None, :], ids, K), axis=0)

    sn_W = sn_W_ref[...]
    sn_b = sn_b_ref[...]
    rwt = rwt_ref[...]
    V = jnp.zeros((K, DS), jnp.float32)
    beta = jnp.zeros((K,), jnp.float32)
    for n in range(NSN):
        V = V + rwt[n][:, None] * sn_W[n]
        beta = beta + rwt[n] * sn_b[n]
    vb_ref[...] = jnp.concatenate(
        [V, beta[None, :], jnp.zeros((VB_ROWS - K - 1, DS), jnp.float32)],
        axis=0)


def _tc1_stage(x, centers, sn_W, sn_b, rw):
    return pl.pallas_call(
        _tc1_body,
        grid=(NBLK,),
        in_specs=[
            pl.BlockSpec((BLK, DX), lambda i: (i, 0)),
            pl.BlockSpec((K, DX), lambda i: (0, 0)),
            pl.BlockSpec((NSN, K, DS), lambda i: (0, 0, 0)),
            pl.BlockSpec((NSN, K), lambda i: (0, 0)),
            pl.BlockSpec((NSN, K), lambda i: (0, 0)),
        ],
        out_specs=[
            pl.BlockSpec((BLK,), lambda i: (i,)),
            pl.BlockSpec((VB_ROWS, DS), lambda i: (0, 0)),
        ],
        out_shape=[
            jax.ShapeDtypeStruct((B,), jnp.int32),
            jax.ShapeDtypeStruct((VB_ROWS, DS), jnp.float32),
        ],
    )(x, centers, sn_W, sn_b, rw)


def _tcbig_body(hidden_ref, tune_W_ref, tune_b_ref, xt_ref):
    h = hidden_ref[...]
    logit = lax.dot_general(tune_W_ref[...], h, (((1,), (1,)), ((), ())),
                            preferred_element_type=jnp.float32)
    xt_ref[...] = jax.nn.sigmoid(logit + tune_b_ref[0, 0]).reshape(-1)


def _tcbig_stage(hidden, tune_W_row, tune_b):
    return pl.pallas_call(
        _tcbig_body,
        grid=(NBLKH,),
        in_specs=[
            pl.BlockSpec((BLKH, DH), lambda i: (i, 0)),
            pl.BlockSpec((1, DH), lambda i: (0, 0)),
            pl.BlockSpec((1, 1), lambda i: (0, 0)),
        ],
        out_specs=[pl.BlockSpec((BLKH,), lambda i: (i,))],
        out_shape=[jax.ShapeDtypeStruct((B,), jnp.float32)],
    )(hidden, tune_W_row, tune_b)


def _blend_body(xsn_ref, xt_ref, np_ref, out_ref):
    xsn = xsn_ref[...]
    xt = xt_ref[...]
    nv = np_ref[...].reshape(-1)
    out_ref[...] = xsn + xt * (nv - xsn)


def _blend_stage(xsn, xt, np_row):
    return pl.pallas_call(
        _blend_body,
        out_shape=jax.ShapeDtypeStruct((B,), jnp.float32),
    )(xsn, xt, np_row)


_NC = 2
_NS = 16
_NW = _NC * _NS
CHUNK = B // _NW
NGROUP = CHUNK // 16
BETA_BASE = K * DS


@functools.lru_cache(maxsize=None)
def _get_sc_stage():
    mesh = plsc.VectorSubcoreMesh(core_axis_name="c", subcore_axis_name="s",
                                  num_cores=_NC, num_subcores=_NS)

    @functools.partial(
        pl.kernel,
        mesh=mesh,
        compiler_params=pltpu.CompilerParams(needs_layout_passes=False),
        out_type=jax.ShapeDtypeStruct((B,), jnp.float32),
        scratch_types=[
            pltpu.VMEM((DS, CHUNK), jnp.float32),
            pltpu.VMEM((VB_ROWS, DS), jnp.float32),
            pltpu.VMEM((CHUNK,), jnp.int32),
            pltpu.VMEM((CHUNK,), jnp.float32),
            pltpu.SemaphoreType.DMA,
            pltpu.SemaphoreType.DMA,
            pltpu.SemaphoreType.DMA,
        ],
    )
    def _sc_stage(st_hbm, vb_hbm, cid_hbm, out_hbm,
                  s_v, vb_v, cid_v, o_v, sem0, sem1, sem2):
        wid = lax.axis_index("s") * _NC + lax.axis_index("c")
        base = wid * CHUNK
        cp0 = pltpu.async_copy(st_hbm.at[:, pl.ds(base, CHUNK)], s_v, sem0)
        cp1 = pltpu.async_copy(vb_hbm, vb_v, sem1)
        cp2 = pltpu.async_copy(cid_hbm.at[pl.ds(base, CHUNK)], cid_v, sem2)
        cp1.wait()
        cp2.wait()
        cp0.wait()

        def group(g, carry):
            t0 = g * 16
            lane = lax.broadcasted_iota(jnp.int32, (16,), 0)
            tok = t0 + lane
            cid = cid_v[pl.ds(t0, 16)]
            acc = plsc.load_gather(vb_v, [jnp.full((16,), K, jnp.int32), cid])
            jvec = lane
            for _ in range(DS):
                sv = plsc.load_gather(s_v, [jvec, tok])
                vv = plsc.load_gather(vb_v, [cid, jvec])
                acc = acc + sv * vv
                jvec = jvec + 1
                jvec = jnp.where(jvec == DS, 0, jvec)
            o_v[pl.ds(t0, 16)] = acc
            return carry

        lax.fori_loop(0, NGROUP, group, 0)
        pltpu.sync_copy(o_v, out_hbm.at[pl.ds(base, CHUNK)])

    return _sc_stage


def kernel(x, s, hidden, naive_pred, centers, tune_W, tune_b, sn_W, sn_b,
           running_sn_weight):
    cid, vb = _tc1_stage(x, centers, sn_W, sn_b,
                         jnp.transpose(running_sn_weight))
    xsn = _get_sc_stage()(jnp.transpose(s), vb, cid)
    [xt] = _tcbig_stage(hidden, tune_W.reshape(1, DH), tune_b.reshape(1, 1))
    out = _blend_stage(xsn, xt, naive_pred.reshape(1, B))
    return out.reshape(-1, 1)


# speedup vs baseline: 1.0852x; 1.0852x over previous
import functools

import jax
import jax.numpy as jnp
from jax import lax
from jax.experimental import pallas as pl
from jax.experimental.pallas import tpu as pltpu
from jax.experimental.pallas import tpu_sc as plsc

B = 8192
K = 64
DX = 128
DS = 64
DH = 1024
NSN = 2

BLK = 2048
NBLK = B // BLK
BLKH = 1024
NBLKH = B // BLKH
VB_ROWS = K + 8


def _tc1_body(x_ref, centers_ref, sn_W_ref, sn_b_ref, rwt_ref,
              cid_ref, vb_ref):
    xb = x_ref[...]
    c = centers_ref[...]
    cs = jnp.sum(c * c, axis=1)
    xc_t = lax.dot_general(c, xb, (((1,), (1,)), ((), ())),
                           preferred_element_type=jnp.float32)
    d2_t = cs[:, None] - 2.0 * xc_t
    m = jnp.min(d2_t, axis=0)
    ids = lax.broadcasted_iota(jnp.int32, d2_t.shape, 0)
    cid_ref[...] = jnp.min(jnp.where(d2_t <= m[None, :], ids, K), axis=0)

    sn_W = sn_W_ref[...]
    sn_b = sn_b_ref[...]
    rwt = rwt_ref[...]
    V = jnp.zeros((K, DS), jnp.float32)
    beta = jnp.zeros((K,), jnp.float32)
    for n in range(NSN):
        V = V + rwt[n][:, None] * sn_W[n]
        beta = beta + rwt[n] * sn_b[n]
    vb_ref[...] = jnp.concatenate(
        [V, beta[None, :], jnp.zeros((VB_ROWS - K - 1, DS), jnp.float32)],
        axis=0)


def _tc1_stage(x, centers, sn_W, sn_b, rw):
    return pl.pallas_call(
        _tc1_body,
        grid=(NBLK,),
        in_specs=[
            pl.BlockSpec((BLK, DX), lambda i: (i, 0)),
            pl.BlockSpec((K, DX), lambda i: (0, 0)),
            pl.BlockSpec((NSN, K, DS), lambda i: (0, 0, 0)),
            pl.BlockSpec((NSN, K), lambda i: (0, 0)),
            pl.BlockSpec((NSN, K), lambda i: (0, 0)),
        ],
        out_specs=[
            pl.BlockSpec((BLK,), lambda i: (i,)),
            pl.BlockSpec((VB_ROWS, DS), lambda i: (0, 0)),
        ],
        out_shape=[
            jax.ShapeDtypeStruct((B,), jnp.int32),
            jax.ShapeDtypeStruct((VB_ROWS, DS), jnp.float32),
        ],
    )(x, centers, sn_W, sn_b, rw)


def _tcbig_body(hidden_ref, tune_W_ref, tune_b_ref, xt_ref):
    h = hidden_ref[...]
    logit = lax.dot_general(tune_W_ref[...], h, (((1,), (1,)), ((), ())),
                            preferred_element_type=jnp.float32)
    xt_ref[...] = jax.nn.sigmoid(logit + tune_b_ref[0, 0]).reshape(-1)


def _tcbig_stage(hidden, tune_W_row, tune_b):
    return pl.pallas_call(
        _tcbig_body,
        grid=(NBLKH,),
        in_specs=[
            pl.BlockSpec((BLKH, DH), lambda i: (i, 0)),
            pl.BlockSpec((1, DH), lambda i: (0, 0)),
            pl.BlockSpec((1, 1), lambda i: (0, 0)),
        ],
        out_specs=[pl.BlockSpec((BLKH,), lambda i: (i,))],
        out_shape=[jax.ShapeDtypeStruct((B,), jnp.float32)],
    )(hidden, tune_W_row, tune_b)


def _blend_body(xsn_ref, xt_ref, np_ref, out_ref):
    xsn = xsn_ref[...]
    xt = xt_ref[...]
    nv = np_ref[...].reshape(-1)
    out_ref[...] = xsn + xt * (nv - xsn)


def _blend_stage(xsn, xt, np_row):
    return pl.pallas_call(
        _blend_body,
        out_shape=jax.ShapeDtypeStruct((B,), jnp.float32),
    )(xsn, xt, np_row)


_NC = 1
_NS = 16
_NW = _NC * _NS
CHUNK = B // _NW
NGROUP = CHUNK // 16
BETA_BASE = K * DS


@functools.lru_cache(maxsize=None)
def _get_sc_stage():
    mesh = plsc.VectorSubcoreMesh(core_axis_name="c", subcore_axis_name="s",
                                  num_cores=_NC, num_subcores=_NS)

    @functools.partial(
        pl.kernel,
        mesh=mesh,
        compiler_params=pltpu.CompilerParams(needs_layout_passes=False),
        out_type=jax.ShapeDtypeStruct((B,), jnp.float32),
        scratch_types=[
            pltpu.VMEM((DS, CHUNK), jnp.float32),
            pltpu.VMEM((VB_ROWS, DS), jnp.float32),
            pltpu.VMEM((CHUNK,), jnp.int32),
            pltpu.VMEM((CHUNK,), jnp.float32),
            pltpu.SemaphoreType.DMA,
            pltpu.SemaphoreType.DMA,
            pltpu.SemaphoreType.DMA,
        ],
    )
    def _sc_stage(st_hbm, vb_hbm, cid_hbm, out_hbm,
                  s_v, vb_v, cid_v, o_v, sem0, sem1, sem2):
        wid = lax.axis_index("s") * _NC + lax.axis_index("c")
        base = wid * CHUNK
        cp0 = pltpu.async_copy(st_hbm.at[:, pl.ds(base, CHUNK)], s_v, sem0)
        cp1 = pltpu.async_copy(vb_hbm, vb_v, sem1)
        cp2 = pltpu.async_copy(cid_hbm.at[pl.ds(base, CHUNK)], cid_v, sem2)
        cp1.wait()
        cp2.wait()
        cp0.wait()

        def group(g, carry):
            t0 = g * 16
            lane = lax.broadcasted_iota(jnp.int32, (16,), 0)
            tok = t0 + lane
            cid = cid_v[pl.ds(t0, 16)]
            acc = plsc.load_gather(vb_v, [jnp.full((16,), K, jnp.int32), cid])
            jvec = lane
            for _ in range(DS):
                sv = plsc.load_gather(s_v, [jvec, tok])
                vv = plsc.load_gather(vb_v, [cid, jvec])
                acc = acc + sv * vv
                jvec = jvec + 1
                jvec = jnp.where(jvec == DS, 0, jvec)
            o_v[pl.ds(t0, 16)] = acc
            return carry

        lax.fori_loop(0, NGROUP, group, 0)
        pltpu.sync_copy(o_v, out_hbm.at[pl.ds(base, CHUNK)])

    return _sc_stage


def kernel(x, s, hidden, naive_pred, centers, tune_W, tune_b, sn_W, sn_b,
           running_sn_weight):
    cid, vb = _tc1_stage(x, centers, sn_W, sn_b,
                         jnp.transpose(running_sn_weight))
    xsn = _get_sc_stage()(jnp.transpose(s), vb, cid)
    [xt] = _tcbig_stage(hidden, tune_W.reshape(1, DH), tune_b.reshape(1, 1))
    out = _blend_stage(xsn, xt, naive_pred.reshape(1, B))
    return out.reshape(-1, 1)


# speedup vs baseline: 1.1055x; 1.0187x over previous
import functools

import jax
import jax.numpy as jnp
from jax import lax
from jax.experimental import pallas as pl
from jax.experimental.pallas import tpu as pltpu
from jax.experimental.pallas import tpu_sc as plsc

B = 8192
K = 64
DX = 128
DS = 64
DH = 1024
NSN = 2

BLK = 4096
NBLK = B // BLK
BLKH = 1024
NBLKH = B // BLKH
VB_ROWS = K + 8


def _tc1_body(x_ref, centers_ref, sn_W_ref, sn_b_ref, rwt_ref,
              cid_ref, vb_ref):
    xb = x_ref[...]
    c = centers_ref[...]
    cs = jnp.sum(c * c, axis=1)
    xc_t = lax.dot_general(c, xb, (((1,), (1,)), ((), ())),
                           preferred_element_type=jnp.float32)
    d2_t = cs[:, None] - 2.0 * xc_t
    m = jnp.min(d2_t, axis=0)
    ids = lax.broadcasted_iota(jnp.int32, d2_t.shape, 0)
    cid_ref[...] = jnp.min(jnp.where(d2_t <= m[None, :], ids, K), axis=0)

    sn_W = sn_W_ref[...]
    sn_b = sn_b_ref[...]
    rwt = rwt_ref[...]
    V = jnp.zeros((K, DS), jnp.float32)
    beta = jnp.zeros((K,), jnp.float32)
    for n in range(NSN):
        V = V + rwt[n][:, None] * sn_W[n]
        beta = beta + rwt[n] * sn_b[n]
    vb_ref[...] = jnp.concatenate(
        [V, beta[None, :], jnp.zeros((VB_ROWS - K - 1, DS), jnp.float32)],
        axis=0)


def _tc1_stage(x, centers, sn_W, sn_b, rw):
    return pl.pallas_call(
        _tc1_body,
        grid=(NBLK,),
        in_specs=[
            pl.BlockSpec((BLK, DX), lambda i: (i, 0)),
            pl.BlockSpec((K, DX), lambda i: (0, 0)),
            pl.BlockSpec((NSN, K, DS), lambda i: (0, 0, 0)),
            pl.BlockSpec((NSN, K), lambda i: (0, 0)),
            pl.BlockSpec((NSN, K), lambda i: (0, 0)),
        ],
        out_specs=[
            pl.BlockSpec((BLK,), lambda i: (i,)),
            pl.BlockSpec((VB_ROWS, DS), lambda i: (0, 0)),
        ],
        out_shape=[
            jax.ShapeDtypeStruct((B,), jnp.int32),
            jax.ShapeDtypeStruct((VB_ROWS, DS), jnp.float32),
        ],
    )(x, centers, sn_W, sn_b, rw)


def _tcbig_body(hidden_ref, tune_W_ref, tune_b_ref, xt_ref):
    h = hidden_ref[...]
    logit = lax.dot_general(tune_W_ref[...], h, (((1,), (1,)), ((), ())),
                            preferred_element_type=jnp.float32)
    xt_ref[...] = jax.nn.sigmoid(logit + tune_b_ref[0, 0]).reshape(-1)


def _tcbig_stage(hidden, tune_W_row, tune_b):
    return pl.pallas_call(
        _tcbig_body,
        grid=(NBLKH,),
        in_specs=[
            pl.BlockSpec((BLKH, DH), lambda i: (i, 0)),
            pl.BlockSpec((1, DH), lambda i: (0, 0)),
            pl.BlockSpec((1, 1), lambda i: (0, 0)),
        ],
        out_specs=[pl.BlockSpec((BLKH,), lambda i: (i,))],
        out_shape=[jax.ShapeDtypeStruct((B,), jnp.float32)],
    )(hidden, tune_W_row, tune_b)


def _blend_body(xsn_ref, xt_ref, np_ref, out_ref):
    xsn = xsn_ref[...]
    xt = xt_ref[...]
    nv = np_ref[...].reshape(-1)
    out_ref[...] = xsn + xt * (nv - xsn)


def _blend_stage(xsn, xt, np_row):
    return pl.pallas_call(
        _blend_body,
        out_shape=jax.ShapeDtypeStruct((B,), jnp.float32),
    )(xsn, xt, np_row)


_NC = 1
_NS = 16
_NW = _NC * _NS
CHUNK = B // _NW
NGROUP = CHUNK // 16
BETA_BASE = K * DS


@functools.lru_cache(maxsize=None)
def _get_sc_stage():
    mesh = plsc.VectorSubcoreMesh(core_axis_name="c", subcore_axis_name="s",
                                  num_cores=_NC, num_subcores=_NS)

    @functools.partial(
        pl.kernel,
        mesh=mesh,
        compiler_params=pltpu.CompilerParams(needs_layout_passes=False),
        out_type=jax.ShapeDtypeStruct((B,), jnp.float32),
        scratch_types=[
            pltpu.VMEM((DS, CHUNK), jnp.float32),
            pltpu.VMEM((VB_ROWS, DS), jnp.float32),
            pltpu.VMEM((CHUNK,), jnp.int32),
            pltpu.VMEM((CHUNK,), jnp.float32),
            pltpu.SemaphoreType.DMA,
            pltpu.SemaphoreType.DMA,
            pltpu.SemaphoreType.DMA,
        ],
    )
    def _sc_stage(st_hbm, vb_hbm, cid_hbm, out_hbm,
                  s_v, vb_v, cid_v, o_v, sem0, sem1, sem2):
        wid = lax.axis_index("s") * _NC + lax.axis_index("c")
        base = wid * CHUNK
        cp0 = pltpu.async_copy(st_hbm.at[:, pl.ds(base, CHUNK)], s_v, sem0)
        cp1 = pltpu.async_copy(vb_hbm, vb_v, sem1)
        cp2 = pltpu.async_copy(cid_hbm.at[pl.ds(base, CHUNK)], cid_v, sem2)
        cp1.wait()
        cp2.wait()
        cp0.wait()

        def group(g, carry):
            t0 = g * 16
            lane = lax.broadcasted_iota(jnp.int32, (16,), 0)
            tok = t0 + lane
            cid = cid_v[pl.ds(t0, 16)]
            acc = plsc.load_gather(vb_v, [jnp.full((16,), K, jnp.int32), cid])
            jvec = lane
            for _ in range(DS):
                sv = plsc.load_gather(s_v, [jvec, tok])
                vv = plsc.load_gather(vb_v, [cid, jvec])
                acc = acc + sv * vv
                jvec = jvec + 1
                jvec = jnp.where(jvec == DS, 0, jvec)
            o_v[pl.ds(t0, 16)] = acc
            return carry

        lax.fori_loop(0, NGROUP, group, 0)
        pltpu.sync_copy(o_v, out_hbm.at[pl.ds(base, CHUNK)])

    return _sc_stage


def kernel(x, s, hidden, naive_pred, centers, tune_W, tune_b, sn_W, sn_b,
           running_sn_weight):
    cid, vb = _tc1_stage(x, centers, sn_W, sn_b,
                         jnp.transpose(running_sn_weight))
    xsn = _get_sc_stage()(jnp.transpose(s), vb, cid)
    [xt] = _tcbig_stage(hidden, tune_W.reshape(1, DH), tune_b.reshape(1, 1))
    out = _blend_stage(xsn, xt, naive_pred.reshape(1, B))
    return out.reshape(-1, 1)


# speedup vs baseline: 1.1063x; 1.0007x over previous
import functools

import jax
import jax.numpy as jnp
from jax import lax
from jax.experimental import pallas as pl
from jax.experimental.pallas import tpu as pltpu
from jax.experimental.pallas import tpu_sc as plsc

B = 8192
K = 64
DX = 128
DS = 64
DH = 1024
NSN = 2

BLK = 4096
NBLK = B // BLK
BLKH = 1024
NBLKH = B // BLKH
VB_ROWS = K + 8


def _tc1_body(x_ref, centers_ref, sn_W_ref, sn_b_ref, rwt_ref,
              cid_ref, vb_ref):
    xb = x_ref[...]
    c = centers_ref[...]
    cs = jnp.sum(c * c, axis=1)
    xc_t = lax.dot_general(c, xb, (((1,), (1,)), ((), ())),
                           preferred_element_type=jnp.float32)
    d2_t = cs[:, None] - 2.0 * xc_t
    m = jnp.min(d2_t, axis=0)
    ids = lax.broadcasted_iota(jnp.int32, d2_t.shape, 0)
    cid_ref[...] = jnp.min(jnp.where(d2_t <= m[None, :], ids, K), axis=0)

    sn_W = sn_W_ref[...]
    sn_b = sn_b_ref[...]
    rwt = rwt_ref[...]
    V = jnp.zeros((K, DS), jnp.float32)
    beta = jnp.zeros((K,), jnp.float32)
    for n in range(NSN):
        V = V + rwt[n][:, None] * sn_W[n]
        beta = beta + rwt[n] * sn_b[n]
    vb_ref[...] = jnp.concatenate(
        [V, beta[None, :], jnp.zeros((VB_ROWS - K - 1, DS), jnp.float32)],
        axis=0)


def _tc1_stage(x, centers, sn_W, sn_b, rw):
    return pl.pallas_call(
        _tc1_body,
        grid=(NBLK,),
        in_specs=[
            pl.BlockSpec((BLK, DX), lambda i: (i, 0)),
            pl.BlockSpec((K, DX), lambda i: (0, 0)),
            pl.BlockSpec((NSN, K, DS), lambda i: (0, 0, 0)),
            pl.BlockSpec((NSN, K), lambda i: (0, 0)),
            pl.BlockSpec((NSN, K), lambda i: (0, 0)),
        ],
        out_specs=[
            pl.BlockSpec((BLK,), lambda i: (i,)),
            pl.BlockSpec((VB_ROWS, DS), lambda i: (0, 0)),
        ],
        out_shape=[
            jax.ShapeDtypeStruct((B,), jnp.int32),
            jax.ShapeDtypeStruct((VB_ROWS, DS), jnp.float32),
        ],
    )(x, centers, sn_W, sn_b, rw)


def _tcbig_body(hidden_ref, tune_W_ref, tune_b_ref, xt_ref):
    h = hidden_ref[...]
    logit = lax.dot_general(tune_W_ref[...], h, (((1,), (1,)), ((), ())),
                            preferred_element_type=jnp.float32)
    xt_ref[...] = jax.nn.sigmoid(logit + tune_b_ref[0, 0]).reshape(-1)


def _tcbig_stage(hidden, tune_W_row, tune_b):
    return pl.pallas_call(
        _tcbig_body,
        grid=(NBLKH,),
        in_specs=[
            pl.BlockSpec((BLKH, DH), lambda i: (i, 0)),
            pl.BlockSpec((1, DH), lambda i: (0, 0)),
            pl.BlockSpec((1, 1), lambda i: (0, 0)),
        ],
        out_specs=[pl.BlockSpec((BLKH,), lambda i: (i,))],
        out_shape=[jax.ShapeDtypeStruct((B,), jnp.float32)],
    )(hidden, tune_W_row, tune_b)


def _blend_body(xsn_ref, xt_ref, np_ref, out_ref):
    xsn = xsn_ref[...]
    xt = xt_ref[...]
    nv = np_ref[...].reshape(-1)
    out_ref[...] = xsn + xt * (nv - xsn)


def _blend_stage(xsn, xt, np_row):
    return pl.pallas_call(
        _blend_body,
        out_shape=jax.ShapeDtypeStruct((B,), jnp.float32),
    )(xsn, xt, np_row)


_NC = 1
_NS = 16
_NW = _NC * _NS
CHUNK = B // _NW
NGROUP = CHUNK // 16
BETA_BASE = K * DS


@functools.lru_cache(maxsize=None)
def _get_sc_stage():
    mesh = plsc.VectorSubcoreMesh(core_axis_name="c", subcore_axis_name="s",
                                  num_cores=_NC, num_subcores=_NS)

    @functools.partial(
        pl.kernel,
        mesh=mesh,
        compiler_params=pltpu.CompilerParams(needs_layout_passes=False,
                                             skip_device_barrier=True),
        out_type=jax.ShapeDtypeStruct((B,), jnp.float32),
        scratch_types=[
            pltpu.VMEM((DS, CHUNK), jnp.float32),
            pltpu.VMEM((VB_ROWS, DS), jnp.float32),
            pltpu.VMEM((CHUNK,), jnp.int32),
            pltpu.VMEM((CHUNK,), jnp.float32),
            pltpu.SemaphoreType.DMA,
            pltpu.SemaphoreType.DMA,
            pltpu.SemaphoreType.DMA,
        ],
    )
    def _sc_stage(st_hbm, vb_hbm, cid_hbm, out_hbm,
                  s_v, vb_v, cid_v, o_v, sem0, sem1, sem2):
        wid = lax.axis_index("s") * _NC + lax.axis_index("c")
        base = wid * CHUNK
        cp0 = pltpu.async_copy(st_hbm.at[:, pl.ds(base, CHUNK)], s_v, sem0)
        cp1 = pltpu.async_copy(vb_hbm, vb_v, sem1)
        cp2 = pltpu.async_copy(cid_hbm.at[pl.ds(base, CHUNK)], cid_v, sem2)
        cp1.wait()
        cp2.wait()
        cp0.wait()

        def group(g, carry):
            t0 = g * 16
            lane = lax.broadcasted_iota(jnp.int32, (16,), 0)
            tok = t0 + lane
            cid = cid_v[pl.ds(t0, 16)]
            acc = plsc.load_gather(vb_v, [jnp.full((16,), K, jnp.int32), cid])
            jvec = lane
            for _ in range(DS):
                sv = plsc.load_gather(s_v, [jvec, tok])
                vv = plsc.load_gather(vb_v, [cid, jvec])
                acc = acc + sv * vv
                jvec = jvec + 1
                jvec = jnp.where(jvec == DS, 0, jvec)
            o_v[pl.ds(t0, 16)] = acc
            return carry

        lax.fori_loop(0, NGROUP, group, 0)
        pltpu.sync_copy(o_v, out_hbm.at[pl.ds(base, CHUNK)])

    return _sc_stage


def kernel(x, s, hidden, naive_pred, centers, tune_W, tune_b, sn_W, sn_b,
           running_sn_weight):
    cid, vb = _tc1_stage(x, centers, sn_W, sn_b,
                         jnp.transpose(running_sn_weight))
    xsn = _get_sc_stage()(jnp.transpose(s), vb, cid)
    [xt] = _tcbig_stage(hidden, tune_W.reshape(1, DH), tune_b.reshape(1, 1))
    out = _blend_stage(xsn, xt, naive_pred.reshape(1, B))
    return out.reshape(-1, 1)


# speedup vs baseline: 1.1079x; 1.0014x over previous
import functools

import jax
import jax.numpy as jnp
from jax import lax
from jax.experimental import pallas as pl
from jax.experimental.pallas import tpu as pltpu
from jax.experimental.pallas import tpu_sc as plsc

B = 8192
K = 64
DX = 128
DS = 64
DH = 1024
NSN = 2

BLK = 8192
NBLK = B // BLK
BLKH = 1024
NBLKH = B // BLKH
VB_ROWS = K + 8


def _tc1_body(x_ref, centers_ref, sn_W_ref, sn_b_ref, rwt_ref,
              cid_ref, vb_ref):
    xb = x_ref[...]
    c = centers_ref[...]
    cs = jnp.sum(c * c, axis=1)
    xc_t = lax.dot_general(c, xb, (((1,), (1,)), ((), ())),
                           preferred_element_type=jnp.float32)
    d2_t = cs[:, None] - 2.0 * xc_t
    m = jnp.min(d2_t, axis=0)
    ids = lax.broadcasted_iota(jnp.int32, d2_t.shape, 0)
    cid_ref[...] = jnp.min(jnp.where(d2_t <= m[None, :], ids, K), axis=0)

    sn_W = sn_W_ref[...]
    sn_b = sn_b_ref[...]
    rwt = rwt_ref[...]
    V = jnp.zeros((K, DS), jnp.float32)
    beta = jnp.zeros((K,), jnp.float32)
    for n in range(NSN):
        V = V + rwt[n][:, None] * sn_W[n]
        beta = beta + rwt[n] * sn_b[n]
    vb_ref[...] = jnp.concatenate(
        [V, beta[None, :], jnp.zeros((VB_ROWS - K - 1, DS), jnp.float32)],
        axis=0)


def _tc1_stage(x, centers, sn_W, sn_b, rw):
    return pl.pallas_call(
        _tc1_body,
        grid=(NBLK,),
        in_specs=[
            pl.BlockSpec((BLK, DX), lambda i: (i, 0)),
            pl.BlockSpec((K, DX), lambda i: (0, 0)),
            pl.BlockSpec((NSN, K, DS), lambda i: (0, 0, 0)),
            pl.BlockSpec((NSN, K), lambda i: (0, 0)),
            pl.BlockSpec((NSN, K), lambda i: (0, 0)),
        ],
        out_specs=[
            pl.BlockSpec((BLK,), lambda i: (i,)),
            pl.BlockSpec((VB_ROWS, DS), lambda i: (0, 0)),
        ],
        out_shape=[
            jax.ShapeDtypeStruct((B,), jnp.int32),
            jax.ShapeDtypeStruct((VB_ROWS, DS), jnp.float32),
        ],
    )(x, centers, sn_W, sn_b, rw)


def _tcbig_body(hidden_ref, tune_W_ref, tune_b_ref, xt_ref):
    h = hidden_ref[...]
    logit = lax.dot_general(tune_W_ref[...], h, (((1,), (1,)), ((), ())),
                            preferred_element_type=jnp.float32)
    xt_ref[...] = jax.nn.sigmoid(logit + tune_b_ref[0, 0]).reshape(-1)


def _tcbig_stage(hidden, tune_W_row, tune_b):
    return pl.pallas_call(
        _tcbig_body,
        grid=(NBLKH,),
        in_specs=[
            pl.BlockSpec((BLKH, DH), lambda i: (i, 0)),
            pl.BlockSpec((1, DH), lambda i: (0, 0)),
            pl.BlockSpec((1, 1), lambda i: (0, 0)),
        ],
        out_specs=[pl.BlockSpec((BLKH,), lambda i: (i,))],
        out_shape=[jax.ShapeDtypeStruct((B,), jnp.float32)],
    )(hidden, tune_W_row, tune_b)


def _blend_body(xsn_ref, xt_ref, np_ref, out_ref):
    xsn = xsn_ref[...]
    xt = xt_ref[...]
    nv = np_ref[...].reshape(-1)
    out_ref[...] = xsn + xt * (nv - xsn)


def _blend_stage(xsn, xt, np_row):
    return pl.pallas_call(
        _blend_body,
        out_shape=jax.ShapeDtypeStruct((B,), jnp.float32),
    )(xsn, xt, np_row)


_NC = 1
_NS = 16
_NW = _NC * _NS
CHUNK = B // _NW
NGROUP = CHUNK // 16
BETA_BASE = K * DS


@functools.lru_cache(maxsize=None)
def _get_sc_stage():
    mesh = plsc.VectorSubcoreMesh(core_axis_name="c", subcore_axis_name="s",
                                  num_cores=_NC, num_subcores=_NS)

    @functools.partial(
        pl.kernel,
        mesh=mesh,
        compiler_params=pltpu.CompilerParams(needs_layout_passes=False),
        out_type=jax.ShapeDtypeStruct((B,), jnp.float32),
        scratch_types=[
            pltpu.VMEM((DS, CHUNK), jnp.float32),
            pltpu.VMEM((VB_ROWS, DS), jnp.float32),
            pltpu.VMEM((CHUNK,), jnp.int32),
            pltpu.VMEM((CHUNK,), jnp.float32),
            pltpu.SemaphoreType.DMA,
            pltpu.SemaphoreType.DMA,
            pltpu.SemaphoreType.DMA,
        ],
    )
    def _sc_stage(st_hbm, vb_hbm, cid_hbm, out_hbm,
                  s_v, vb_v, cid_v, o_v, sem0, sem1, sem2):
        wid = lax.axis_index("s") * _NC + lax.axis_index("c")
        base = wid * CHUNK
        cp0 = pltpu.async_copy(st_hbm.at[:, pl.ds(base, CHUNK)], s_v, sem0)
        cp1 = pltpu.async_copy(vb_hbm, vb_v, sem1)
        cp2 = pltpu.async_copy(cid_hbm.at[pl.ds(base, CHUNK)], cid_v, sem2)
        cp1.wait()
        cp2.wait()
        cp0.wait()

        def group(g, carry):
            t0 = g * 16
            lane = lax.broadcasted_iota(jnp.int32, (16,), 0)
            tok = t0 + lane
            cid = cid_v[pl.ds(t0, 16)]
            acc = plsc.load_gather(vb_v, [jnp.full((16,), K, jnp.int32), cid])
            jvec = lane
            for _ in range(DS):
                sv = plsc.load_gather(s_v, [jvec, tok])
                vv = plsc.load_gather(vb_v, [cid, jvec])
                acc = acc + sv * vv
                jvec = jvec + 1
                jvec = jnp.where(jvec == DS, 0, jvec)
            o_v[pl.ds(t0, 16)] = acc
            return carry

        lax.fori_loop(0, NGROUP, group, 0)
        pltpu.sync_copy(o_v, out_hbm.at[pl.ds(base, CHUNK)])

    return _sc_stage


def kernel(x, s, hidden, naive_pred, centers, tune_W, tune_b, sn_W, sn_b,
           running_sn_weight):
    cid, vb = _tc1_stage(x, centers, sn_W, sn_b,
                         jnp.transpose(running_sn_weight))
    xsn = _get_sc_stage()(jnp.transpose(s), vb, cid)
    [xt] = _tcbig_stage(hidden, tune_W.reshape(1, DH), tune_b.reshape(1, 1))
    out = _blend_stage(xsn, xt, naive_pred.reshape(1, B))
    return out.reshape(-1, 1)
